# Initial kernel scaffold; baseline (speedup 1.0000x reference)
import jax
import jax.numpy as jnp
from jax.experimental import pallas as pl


def kernel(upscaled_feats, grouping_idx, grouping_point_mask, W_qkv, W_proj, b_proj, gamma):
    raise NotImplementedError("write your pallas kernel here")


# speedup vs baseline: 1.9273x; 1.9273x over previous
import functools

import jax
import jax.numpy as jnp
from jax import lax
from jax.experimental import pallas as pl
from jax.experimental.pallas import tpu as pltpu
from jax.experimental.pallas import tpu_sc as plsc

B, N_MAX, C = 2, 16384, 256
G, K = 512, 64
H = 8
HD = C // H
SCALE = HD ** -0.5

BG = B * G
R = B * G * K
RB = 128
NW = 32
ROWS_PER_W = R // NW
CW = 256
CNT_PAD = 256

def _gather_body(idx_hbm, feats_hbm, out_hbm, idx_v, rows_v, sem):
    cid = lax.axis_index("c")
    sid = lax.axis_index("s")
    wid = sid * 2 + cid
    base = wid * ROWS_PER_W
    off = jnp.where(base >= R // 2, N_MAX, 0).astype(jnp.int32)

    def body(i, _):
        r0 = base + i * RB
        pltpu.sync_copy(idx_hbm.at[pl.ds(r0, RB)], idx_v)
        for j in range(RB // 16):
            idx_v[pl.ds(j * 16, 16)] = idx_v[pl.ds(j * 16, 16)] + off
        pltpu.async_copy(feats_hbm.at[idx_v], rows_v, sem).wait()
        pltpu.sync_copy(rows_v, out_hbm.at[pl.ds(r0, RB)])
        return 0

    lax.fori_loop(0, ROWS_PER_W // RB, body, 0)


def _gather_sc(idx_flat, feats_flat):
    mesh = plsc.VectorSubcoreMesh(core_axis_name="c", subcore_axis_name="s")
    fn = pl.kernel(
        _gather_body,
        mesh=mesh,
        out_type=jax.ShapeDtypeStruct((R, C), jnp.float32),
        scratch_types=[
            pltpu.VMEM((RB,), jnp.int32),
            pltpu.VMEM((RB, C), jnp.float32),
            pltpu.SemaphoreType.DMA,
        ],
    )
    return fn(idx_flat, feats_flat)


def _attn_body(xg_ref, mb_ref, wqkv_ref, wproj_ref, bproj_ref, out_ref):
    f32 = jnp.float32
    x = xg_ref[...]
    mb = mb_ref[0]

    qkv = jnp.dot(x, wqkv_ref[...], preferred_element_type=f32)
    q = qkv[:, :C] * SCALE
    k = qkv[:, C:2 * C]
    v = qkv[:, 2 * C:]

    r_i = lax.broadcasted_iota(jnp.int32, (H * K, C), 0)
    c_i = lax.broadcasted_iota(jnp.int32, (H * K, C), 1)
    M = (r_i // K == c_i // HD).astype(f32)

    ktile = jnp.concatenate([k] * H, axis=0)
    K2 = ktile * M
    logits = lax.dot_general(q, K2, (((1,), (1,)), ((), ())),
                             preferred_element_type=f32)
    e = jnp.exp(logits + mb)

    br = lax.broadcasted_iota(jnp.int32, (H * K, H), 0)
    bc = lax.broadcasted_iota(jnp.int32, (H * K, H), 1)
    blk = (br // K == bc).astype(f32)
    s = jnp.dot(e, blk, preferred_element_type=f32)
    r = 1.0 / jnp.maximum(s, 1e-30)
    rexp = lax.dot_general(r, blk, (((1,), (1,)), ((), ())),
                           preferred_element_type=f32)
    attn = e * rexp

    vtile = jnp.concatenate([v] * H, axis=0)
    V2 = vtile * M
    av = jnp.dot(attn, V2, preferred_element_type=f32)
    out = jnp.dot(av, wproj_ref[...], preferred_element_type=f32) + bproj_ref[...]

    m01 = jnp.where(mb[:, :K] < -1.0, 0.0, 1.0)
    di = lax.broadcasted_iota(jnp.int32, (K, K), 0)
    dj = lax.broadcasted_iota(jnp.int32, (K, K), 1)
    D = (di == dj).astype(f32) * m01
    out_ref[...] = jnp.dot(D, out, preferred_element_type=f32)


def _attn_tc(xg, mask_bias, W_qkv, W_proj, b_proj2):
    return pl.pallas_call(
        _attn_body,
        grid=(BG,),
        in_specs=[
            pl.BlockSpec((K, C), lambda g: (g, 0)),
            pl.BlockSpec((1, 1, H * K), lambda g: (g, 0, 0)),
            pl.BlockSpec((C, 3 * C), lambda g: (0, 0)),
            pl.BlockSpec((C, C), lambda g: (0, 0)),
            pl.BlockSpec((1, C), lambda g: (0, 0)),
        ],
        out_specs=pl.BlockSpec((K, C), lambda g: (g, 0)),
        out_shape=jax.ShapeDtypeStruct((R, C), jnp.float32),
    )(xg, mask_bias, W_qkv, W_proj, b_proj2)


def _scatter_body(idx_hbm, mask_hbm, upd_hbm,
                  acc_out, cnt_out,
                  idx_v, tgt_v, mask_v, rows_v, ones_v):
    cid = lax.axis_index("c")
    sid = lax.axis_index("s")
    rows_per_batch = R // B

    def zinit(i, _):
        for j in range(C // 16):
            rows_v[i, pl.ds(j * 16, 16)] = jnp.zeros((16,), jnp.float32)
        for j in range(CW // 16):
            ones_v[i, pl.ds(j * 16, 16)] = jnp.zeros((16,), jnp.float32)
        return 0

    lax.fori_loop(0, RB, zinit, 0)
    zbase = cid * N_MAX + sid * (N_MAX // 16)

    def zslab(s, _):
        pltpu.sync_copy(rows_v, acc_out.at[pl.ds(zbase + s * RB, RB)])
        pltpu.sync_copy(ones_v, cnt_out.at[pl.ds(zbase + s * RB, RB)])
        return 0

    lax.fori_loop(0, N_MAX // 16 // RB, zslab, 0)
    dump0 = B * N_MAX + cid * 128

    @pl.when(sid == 0)
    def _():
        pltpu.sync_copy(ones_v, cnt_out.at[pl.ds(dump0, RB)])

    def oinit(i, _):
        for j in range(CW // 16):
            ones_v[i, pl.ds(j * 16, 16)] = jnp.ones((16,), jnp.float32)
        return 0

    lax.fori_loop(0, RB, oinit, 0)
    plsc.subcore_barrier()

    rowbase = cid * rows_per_batch + sid * (rows_per_batch // 16)
    tgt_off = cid * N_MAX
    lane = lax.iota(jnp.int32, 16)

    def sbody(blk, _):
        r0 = rowbase + blk * RB
        pltpu.sync_copy(idx_hbm.at[pl.ds(r0, RB)], idx_v)
        pltpu.sync_copy(mask_hbm.at[pl.ds(r0, RB)], mask_v)
        for j in range(RB // 16):
            tgt = idx_v[pl.ds(j * 16, 16)] + tgt_off
            m = mask_v[pl.ds(j * 16, 16)]
            idx_v[pl.ds(j * 16, 16)] = tgt
            tgt_v[pl.ds(j * 16, 16)] = jnp.where(
                m > 0, tgt, dump0 + j * 16 + lane)
        pltpu.sync_copy(upd_hbm.at[pl.ds(r0, RB)], rows_v)
        pltpu.sync_copy(rows_v, acc_out.at[idx_v], add=True)
        pltpu.sync_copy(ones_v, cnt_out.at[tgt_v], add=True)
        return 0

    lax.fori_loop(0, rows_per_batch // 16 // RB, sbody, 0)


def _scatter_sc(idx_flat, maskf_flat, updated):
    mesh = plsc.VectorSubcoreMesh(core_axis_name="c", subcore_axis_name="s")
    fn = pl.kernel(
        _scatter_body,
        mesh=mesh,
        out_type=(
            jax.ShapeDtypeStruct((B * N_MAX, C), jnp.float32),
            jax.ShapeDtypeStruct((B * N_MAX + CNT_PAD, CW), jnp.float32),
        ),
        scratch_types=[
            pltpu.VMEM((RB,), jnp.int32),
            pltpu.VMEM((RB,), jnp.int32),
            pltpu.VMEM((RB,), jnp.float32),
            pltpu.VMEM((RB, C), jnp.float32),
            pltpu.VMEM((RB, CW), jnp.float32),
        ],
    )
    return fn(idx_flat, maskf_flat, updated)


def _combine_body(up_ref, acc_ref, cnt_ref, gamma_ref, out_ref):
    cnt = jnp.max(cnt_ref[...], axis=1, keepdims=True)
    denom = jnp.maximum(cnt, 1.0)
    out_ref[...] = up_ref[...] + acc_ref[...] * gamma_ref[...] / denom


def _combine_tc(up, acc, cnt, gamma2):
    BLK = 512
    return pl.pallas_call(
        _combine_body,
        grid=(B * N_MAX // BLK,),
        in_specs=[
            pl.BlockSpec((BLK, C), lambda g: (g, 0)),
            pl.BlockSpec((BLK, C), lambda g: (g, 0)),
            pl.BlockSpec((BLK, CW), lambda g: (g, 0)),
            pl.BlockSpec((1, C), lambda g: (0, 0)),
        ],
        out_specs=pl.BlockSpec((BLK, C), lambda g: (g, 0)),
        out_shape=jax.ShapeDtypeStruct((B * N_MAX, C), jnp.float32),
    )(up, acc, cnt, gamma2)


def kernel(upscaled_feats, grouping_idx, grouping_point_mask, W_qkv, W_proj,
           b_proj, gamma):
    idx = jnp.where(grouping_idx < 0, 0, grouping_idx).astype(jnp.int32)
    idx_flat = idx.reshape(R)
    feats_flat = upscaled_feats.reshape(B * N_MAX, C)

    xg = _gather_sc(idx_flat, feats_flat)

    maskf = grouping_point_mask.astype(jnp.float32)
    mask_bias = jnp.tile((maskf.reshape(BG, 1, K) - 1.0) * 1e5, (1, 1, H))
    updated = _attn_tc(xg, mask_bias, W_qkv, W_proj, b_proj.reshape(1, C))

    acc, cntp = _scatter_sc(idx_flat, maskf.reshape(R), updated)

    out = _combine_tc(feats_flat, acc, cntp[:B * N_MAX], gamma.reshape(1, C))
    return out.reshape(B, N_MAX, C)
